# baseline (speedup 1.0000x reference)
import jax
import jax.numpy as jnp
from jax import lax
from jax.experimental import pallas as pl
from jax.experimental.pallas import tpu as pltpu
from jax.experimental.pallas import tpu_sc as plsc

VOCAB = 1000000
D = 32
ROWS = 16384
COLS = 200
B = ROWS * COLS
NC = 2
NS = 16
NW = NC * NS
ITPW = 4
C = ITPW * 128
NJ = COLS
NBUF = 3
SCALE = float(D) ** 0.5


def _body(xT_hbm, table_hbm, out_hbm, idx_v, rows_v, isem, gsem, wsem):
    wid = lax.axis_index("s") * NC + lax.axis_index("c")
    colbase = wid * C

    def idx_copy(j, b):
        return pltpu.make_async_copy(
            xT_hbm.at[j, pl.ds(colbase, C)], idx_v.at[b], isem.at[b]
        )

    def gather_copies(b, ib):
        return [
            pltpu.make_async_copy(
                table_hbm.at[idx_v.at[ib, pl.ds(t * 128, 128)]],
                rows_v.at[b, t],
                gsem.at[b],
            )
            for t in range(ITPW)
        ]

    def write_copies(j, b):
        return [
            pltpu.make_async_copy(
                rows_v.at[b, :, :, d],
                out_hbm.at[j, d // 8, pl.ds(wid * ITPW, ITPW), d % 8, :],
                wsem.at[b],
            )
            for d in range(D)
        ]

    idx_copy(0, 0).start()
    idx_copy(0, 0).wait()
    for c in gather_copies(0, 0):
        c.start()
    idx_copy(1, 1).start()

    @pl.loop(0, NJ)
    def _chunk(j):
        b = lax.rem(j, NBUF)
        ib = lax.rem(j, 2)
        nb = lax.rem(j + 1, NBUF)

        @pl.when(j + 1 < NJ)
        def _():
            @pl.when(j + 1 >= NBUF)
            def _():
                for c in write_copies(j + 1 - NBUF, nb):
                    c.wait()

            idx_copy(j + 1, 1 - ib).wait()
            for c in gather_copies(nb, 1 - ib):
                c.start()

        for c in gather_copies(b, ib):
            c.wait()

        @pl.when(j + 2 < NJ)
        def _():
            idx_copy(j + 2, ib).start()

        rv = rows_v.at[b]

        @pl.loop(0, 4)
        def _t(t):
            @pl.loop(0, 128, unroll=8)
            def _ic(ic):
                rv[t, ic, pl.ds(0, 16)] = rv[t, ic, pl.ds(0, 16)] * SCALE
                rv[t, ic, pl.ds(16, 16)] = rv[t, ic, pl.ds(16, 16)] * SCALE

        for c in write_copies(j, b):
            c.start()

    for jt in range(NJ - NBUF, NJ):
        for c in write_copies(jt, jt % NBUF):
            c.wait()


@jax.jit
def _embed(xT, table):
    mesh = plsc.VectorSubcoreMesh(
        core_axis_name="c", subcore_axis_name="s", num_cores=NC, num_subcores=NS
    )
    out5 = pl.kernel(
        _body,
        out_type=jax.ShapeDtypeStruct((NJ, 4, 128, 8, 128), jnp.float32),
        mesh=mesh,
        compiler_params=pltpu.CompilerParams(
            use_tc_tiling_on_sc=False, needs_layout_passes=False
        ),
        scratch_types=[
            pltpu.VMEM((2, C), jnp.int32),
            pltpu.VMEM((NBUF, ITPW, 128, D), jnp.float32),
            pltpu.SemaphoreType.DMA((2,)),
            pltpu.SemaphoreType.DMA((NBUF,)),
            pltpu.SemaphoreType.DMA((NBUF,)),
        ],
    )(xT.astype(jnp.int32), table)
    return out5.transpose(2, 4, 0, 1, 3).reshape(ROWS, COLS, D)


def kernel(x, table):
    return _embed(x.T, table)


# speedup vs baseline: 199.6367x; 199.6367x over previous
import jax
import jax.numpy as jnp
from jax import lax
from jax.experimental import pallas as pl
from jax.experimental.pallas import tpu as pltpu
from jax.experimental.pallas import tpu_sc as plsc

VOCAB = 1000000
D = 32
ROWS = 16384
COLS = 200
B = ROWS * COLS
NC = 2
NS = 16
NW = NC * NS
ITPW = 4
C = ITPW * 128
NJ = COLS
PITCH = 131
SCALE = float(D) ** 0.5


def _body(xT_hbm, table_hbm, out_hbm, idx_v, rows_v, tbuf, isem, gsem, wsem):
    wid = lax.axis_index("s") * NC + lax.axis_index("c")
    colbase = wid * C
    i16 = lax.iota(jnp.int32, 16)

    def idx_copy(j, b):
        return pltpu.make_async_copy(
            xT_hbm.at[j, pl.ds(colbase, C)], idx_v.at[b], isem.at[b]
        )

    def gather_copy(b):
        return pltpu.make_async_copy(
            table_hbm.at[idx_v.at[b]], rows_v.at[b], gsem.at[b]
        )

    def write_copy(j, b):
        return pltpu.make_async_copy(
            tbuf.at[b, :, :, :, pl.ds(0, 128)],
            out_hbm.at[j, :, pl.ds(wid * ITPW, ITPW)],
            wsem.at[b],
        )

    idx_copy(0, 0).start()
    idx_copy(0, 0).wait()
    gather_copy(0).start()
    idx_copy(1, 1).start()

    @pl.loop(0, NJ)
    def _chunk(j):
        b = lax.rem(j, 2)
        nb = 1 - b

        @pl.when(j + 1 < NJ)
        def _():
            idx_copy(j + 1, nb).wait()
            gather_copy(nb).start()

        gather_copy(b).wait()

        @pl.when(j + 2 < NJ)
        def _():
            idx_copy(j + 2, b).start()

        @pl.when(j >= 2)
        def _():
            write_copy(j - 2, b).wait()

        rv = rows_v.at[b]
        tb = tbuf.at[b]

        dt0 = lax.shift_right_logical(i16, 3)
        r0 = lax.bitwise_and(i16, 7)
        dt1 = dt0 + 2
        for t in range(ITPW):
            tv = jnp.full((16,), t, jnp.int32)

            @pl.loop(0, 128, unroll=8)
            def _ic(ic):
                row = t * 128 + ic
                v0 = rv[row, pl.ds(0, 16)] * SCALE
                v1 = rv[row, pl.ds(16, 16)] * SCALE
                icv = jnp.full((16,), 0, jnp.int32) + ic
                plsc.store_scatter(tb, [dt0, tv, r0, icv], v0)
                plsc.store_scatter(tb, [dt1, tv, r0, icv], v1)

        write_copy(j, b).start()

    write_copy(NJ - 2, 0).wait()
    write_copy(NJ - 1, 1).wait()


@jax.jit
def _embed(xT, table):
    mesh = plsc.VectorSubcoreMesh(
        core_axis_name="c", subcore_axis_name="s", num_cores=NC, num_subcores=NS
    )
    out5 = pl.kernel(
        _body,
        out_type=jax.ShapeDtypeStruct((NJ, 4, 128, 8, 128), jnp.float32),
        mesh=mesh,
        compiler_params=pltpu.CompilerParams(
            use_tc_tiling_on_sc=False, needs_layout_passes=False
        ),
        scratch_types=[
            pltpu.VMEM((2, C), jnp.int32),
            pltpu.VMEM((2, C, D), jnp.float32),
            pltpu.VMEM((2, 4, ITPW, 8, PITCH), jnp.float32),
            pltpu.SemaphoreType.DMA((2,)),
            pltpu.SemaphoreType.DMA((2,)),
            pltpu.SemaphoreType.DMA((2,)),
        ],
    )(xT.astype(jnp.int32), table)
    return out5.transpose(2, 4, 0, 1, 3).reshape(ROWS, COLS, D)


def kernel(x, table):
    return _embed(x.T, table)
